# baseline (device time: 40660 ns/iter reference)
import functools

import jax
import jax.numpy as jnp
from jax import lax
from jax.experimental import pallas as pl
from jax.experimental.pallas import tpu as pltpu

N_DEV = 8
K = 16
N_ROUNDS = 3
NEG_INF = float("-inf")


def _topk_desc(x, k):
    m = jnp.max(x, axis=1, keepdims=True)
    cols = [m]
    for _ in range(k - 1):
        m = jnp.max(jnp.where(x < m, x, NEG_INF), axis=1, keepdims=True)
        cols.append(m)
    return jnp.concatenate(cols, axis=1)


def kernel(x):
    m, n = x.shape

    def body(x_ref, out_ref, cand_ref, recv_ref, send_sems, recv_sems):
        my_id = lax.axis_index("i")
        partners = [my_id ^ (1 << r) for r in range(N_ROUNDS)]

        barrier = pltpu.get_barrier_semaphore()
        for p in partners:
            pl.semaphore_signal(
                barrier, inc=1,
                device_id=(p,), device_id_type=pl.DeviceIdType.MESH,
            )
        pl.semaphore_wait(barrier, N_ROUNDS)

        cand_ref[:, :] = _topk_desc(x_ref[:, :], K)

        for r in range(N_ROUNDS):
            rdma = pltpu.make_async_remote_copy(
                src_ref=cand_ref,
                dst_ref=recv_ref.at[r],
                send_sem=send_sems.at[r],
                recv_sem=recv_sems.at[r],
                device_id=(partners[r],),
                device_id_type=pl.DeviceIdType.MESH,
            )
            rdma.start()
            rdma.wait()
            merged = jnp.concatenate(
                [cand_ref[:, :], recv_ref[r, :, :]], axis=1
            )
            cand_ref[:, :] = _topk_desc(merged, K)

        out_ref[:, :] = cand_ref[:, :]

        @functools.partial(
            pl.run_scoped, second_barrier=pltpu.SemaphoreType.REGULAR
        )
        def _(second_barrier):
            for p in partners:
                pl.semaphore_signal(
                    second_barrier, inc=1,
                    device_id=(p,), device_id_type=pl.DeviceIdType.MESH,
                )
            pl.semaphore_wait(second_barrier, N_ROUNDS)

    return pl.pallas_call(
        body,
        out_shape=jax.ShapeDtypeStruct((m, K), jnp.float32),
        in_specs=[pl.BlockSpec(memory_space=pltpu.VMEM)],
        out_specs=pl.BlockSpec(memory_space=pltpu.VMEM),
        scratch_shapes=[
            pltpu.VMEM((m, K), jnp.float32),
            pltpu.VMEM((N_ROUNDS, m, K), jnp.float32),
            pltpu.SemaphoreType.DMA((N_ROUNDS,)),
            pltpu.SemaphoreType.DMA((N_ROUNDS,)),
        ],
        compiler_params=pltpu.CompilerParams(collective_id=0),
    )(x)


# device time: 17008 ns/iter; 2.3906x vs baseline; 2.3906x over previous
import functools

import jax
import jax.numpy as jnp
from jax import lax
from jax.experimental import pallas as pl
from jax.experimental.pallas import tpu as pltpu

N_DEV = 8
K = 16
N_ROUNDS = 3
NEG_INF = float("-inf")


def _topk_iter(x, k):
    m = jnp.max(x, axis=1, keepdims=True)
    cols = [m]
    for _ in range(k - 1):
        m = jnp.max(jnp.where(x < m, x, NEG_INF), axis=1, keepdims=True)
        cols.append(m)
    return jnp.concatenate(cols, axis=1)


def _topk_desc(x, k):
    n = x.shape[1]
    if k <= 1 or n <= 256 or 2 * k >= n:
        return _topk_iter(x, k)
    h = n // 2
    lo, hi = x[:, :h], x[:, h:]
    p = jnp.maximum(lo, hi)
    q = jnp.minimum(lo, hi)
    cand = jnp.concatenate(
        [_topk_desc(p, k), _topk_desc(q, k // 2)], axis=1
    )
    return _topk_iter(cand, k)


def kernel(x):
    m, n = x.shape

    def body(x_ref, out_ref, cand_ref, recv_ref, send_sems, recv_sems):
        my_id = lax.axis_index("i")
        partners = [my_id ^ (1 << r) for r in range(N_ROUNDS)]

        barrier = pltpu.get_barrier_semaphore()
        for p in partners:
            pl.semaphore_signal(
                barrier, inc=1,
                device_id=(p,), device_id_type=pl.DeviceIdType.MESH,
            )
        pl.semaphore_wait(barrier, N_ROUNDS)

        cand_ref[:, :] = _topk_desc(x_ref[:, :], K)

        for r in range(N_ROUNDS):
            rdma = pltpu.make_async_remote_copy(
                src_ref=cand_ref,
                dst_ref=recv_ref.at[r],
                send_sem=send_sems.at[r],
                recv_sem=recv_sems.at[r],
                device_id=(partners[r],),
                device_id_type=pl.DeviceIdType.MESH,
            )
            rdma.start()
            rdma.wait()
            merged = jnp.concatenate(
                [cand_ref[:, :], recv_ref[r, :, :]], axis=1
            )
            cand_ref[:, :] = _topk_desc(merged, K)

        out_ref[:, :] = cand_ref[:, :]

        @functools.partial(
            pl.run_scoped, second_barrier=pltpu.SemaphoreType.REGULAR
        )
        def _(second_barrier):
            for p in partners:
                pl.semaphore_signal(
                    second_barrier, inc=1,
                    device_id=(p,), device_id_type=pl.DeviceIdType.MESH,
                )
            pl.semaphore_wait(second_barrier, N_ROUNDS)

    return pl.pallas_call(
        body,
        out_shape=jax.ShapeDtypeStruct((m, K), jnp.float32),
        in_specs=[pl.BlockSpec(memory_space=pltpu.VMEM)],
        out_specs=pl.BlockSpec(memory_space=pltpu.VMEM),
        scratch_shapes=[
            pltpu.VMEM((m, K), jnp.float32),
            pltpu.VMEM((N_ROUNDS, m, K), jnp.float32),
            pltpu.SemaphoreType.DMA((N_ROUNDS,)),
            pltpu.SemaphoreType.DMA((N_ROUNDS,)),
        ],
        compiler_params=pltpu.CompilerParams(collective_id=0),
    )(x)
